# baseline (device time: 34447 ns/iter reference)
import jax
import jax.numpy as jnp
from jax import lax
from jax.experimental import pallas as pl
from jax.experimental.pallas import tpu as pltpu

N_DEV = 16
BLK = 256


def kernel(x, w_mat, scale_x, scale_w):
    m_per, k = x.shape
    _, n = w_mat.shape
    n_per = n // N_DEV
    n_blocks = n // BLK

    def body(sx_ref, sw_ref, x_ref, w_ref, out_ref,
             x8_ref, wblk_ref, sbuf_ref, rbuf_ref,
             wdma_sems, send_sems, recv_sems):
        my = lax.axis_index("i")
        pair = my // 2
        s = sx_ref[0] * sw_ref[0]

        x8_ref[...] = x_ref[...].astype(jnp.float8_e4m3fn)

        def start_wdma(t):
            c = (pair + t) % n_blocks
            cp = pltpu.make_async_copy(
                w_ref.at[:, pl.ds(c * BLK, BLK)],
                wblk_ref.at[t % 2],
                wdma_sems.at[t % 2],
            )
            cp.start()
            return cp

        cp_prev = start_wdma(0)
        for t in range(n_blocks):
            cp_cur = cp_prev
            if t + 1 < n_blocks:
                cp_prev = start_wdma(t + 1)
            cp_cur.wait()

            c = (pair + t) % n_blocks
            w8 = wblk_ref[t % 2].astype(jnp.float8_e4m3fn)
            acc = jnp.dot(x8_ref[...], w8, preferred_element_type=jnp.float32)
            y32 = jnp.maximum(acc * s, 0.0)

            for b in range(2):
                tgt = 2 * c + b
                chunk = y32[:, b * n_per:(b + 1) * n_per]

                @pl.when(tgt == my)
                def _(chunk=chunk):
                    out_ref[pl.ds(my * m_per, m_per), :] = chunk

                @pl.when(tgt != my)
                def _(chunk=chunk, tgt=tgt):
                    sbuf_ref[tgt] = chunk.astype(jnp.bfloat16)
                    rdma = pltpu.make_async_remote_copy(
                        src_ref=sbuf_ref.at[tgt],
                        dst_ref=rbuf_ref.at[my],
                        send_sem=send_sems.at[tgt],
                        recv_sem=recv_sems.at[my],
                        device_id=(tgt,),
                        device_id_type=pl.DeviceIdType.MESH,
                    )
                    rdma.start()

        for t in range(n_blocks):
            for b in range(2):
                src = 2 * ((pair - t + n_blocks) % n_blocks) + b

                @pl.when(src != my)
                def _(src=src):
                    recv = pltpu.make_async_remote_copy(
                        src_ref=rbuf_ref.at[src],
                        dst_ref=rbuf_ref.at[src],
                        send_sem=send_sems.at[src],
                        recv_sem=recv_sems.at[src],
                        device_id=(src,),
                        device_id_type=pl.DeviceIdType.MESH,
                    )
                    recv.wait_recv()
                    out_ref[pl.ds(src * m_per, m_per), :] = (
                        rbuf_ref[src].astype(jnp.float32)
                    )

        for off in range(1, N_DEV):
            tgt = (my + off) % N_DEV

            send = pltpu.make_async_remote_copy(
                src_ref=sbuf_ref.at[tgt],
                dst_ref=rbuf_ref.at[my],
                send_sem=send_sems.at[tgt],
                recv_sem=recv_sems.at[my],
                device_id=(tgt,),
                device_id_type=pl.DeviceIdType.MESH,
            )
            send.wait_send()

    return pl.pallas_call(
        body,
        out_shape=jax.ShapeDtypeStruct((N_DEV * m_per, n_per), jnp.float32),
        in_specs=[
            pl.BlockSpec(memory_space=pltpu.SMEM),
            pl.BlockSpec(memory_space=pltpu.SMEM),
            pl.BlockSpec(memory_space=pltpu.VMEM),
            pl.BlockSpec(memory_space=pl.ANY),
        ],
        out_specs=pl.BlockSpec(memory_space=pltpu.VMEM),
        scratch_shapes=[
            pltpu.VMEM((m_per, k), jnp.float8_e4m3fn),
            pltpu.VMEM((2, k, BLK), jnp.float32),
            pltpu.VMEM((N_DEV, m_per, n_per), jnp.bfloat16),
            pltpu.VMEM((N_DEV, m_per, n_per), jnp.bfloat16),
            pltpu.SemaphoreType.DMA((2,)),
            pltpu.SemaphoreType.DMA((N_DEV,)),
            pltpu.SemaphoreType.DMA((N_DEV,)),
        ],
        compiler_params=pltpu.CompilerParams(
            vmem_limit_bytes=56 * 1024 * 1024,
        ),
    )(scale_x, scale_w, x, w_mat)
